# baseline (speedup 1.0000x reference)
import functools

import jax
import jax.numpy as jnp
from jax import lax
from jax.experimental import pallas as pl
from jax.experimental.pallas import tpu as pltpu
from jax.experimental.pallas import tpu_sc as plsc

_N = 10000
_E = 160000
_D = 256
_NT = 32
_RA = 320
_RB = 312
_SB = 2000
_NB = _E // _SB
_NG = _SB // 16
_K = 80
_PEND = _SB + 16
_SHIFT = 14
_MASKLO = (1 << _SHIFT) - 1


def _mm_body(a_ref, x_ref, w_ref, b_ref, o_ref, *, prelu):
    x = x_ref[...]
    if prelu:
        al = a_ref[0]
        x = jnp.where(x >= 0, x, al * x)
    o_ref[...] = (
        jnp.dot(x, w_ref[...], preferred_element_type=jnp.float32) + b_ref[...]
    )


def _mm(x, w, b, a, prelu):
    blk = 2000
    grid = (_N // blk,)
    return pl.pallas_call(
        functools.partial(_mm_body, prelu=prelu),
        grid=grid,
        in_specs=[
            pl.BlockSpec(memory_space=pltpu.SMEM),
            pl.BlockSpec((blk, _D), lambda i: (i, 0)),
            pl.BlockSpec((_D, _D), lambda i: (0, 0)),
            pl.BlockSpec((1, _D), lambda i: (0, 0)),
        ],
        out_specs=pl.BlockSpec((blk, _D), lambda i: (i, 0)),
        out_shape=jax.ShapeDtypeStruct((_N, _D), jnp.float32),
    )(a, x, w, b.reshape(1, _D))


def _spmm_body(h_hbm, src_hbm, dst_hbm, w_hbm, out_hbm,
               src_st, dst_st, w_st, pend_pk, pend_w,
               srcb, dlb, wb, rows_v, acc, sem):
    c = lax.axis_index("c")
    s = lax.axis_index("s")
    g = c * 16 + s
    row0 = g * _RB + jnp.minimum(g, 2) * (_RA - _RB)
    nrows = jnp.where(g < 2, _RA, _RB)

    def _zero(r, carry):
        for j in range(_D // 16):
            acc[r, pl.ds(j * 16, 16)] = jnp.zeros((16,), jnp.float32)
        return carry

    lax.fori_loop(0, _RA, _zero, 0)

    zero16 = jnp.zeros((16,), jnp.float32)
    iota16 = lax.iota(jnp.int32, 16)

    def block(b, carry):
        eb = b * _SB
        pltpu.sync_copy(src_hbm.at[pl.ds(eb, _SB)], src_st)
        pltpu.sync_copy(dst_hbm.at[pl.ds(eb, _SB)], dst_st)
        pltpu.sync_copy(w_hbm.at[pl.ds(eb, _SB)], w_st)

        def scan(i, off):
            base = i * 16
            d16 = dst_st[pl.ds(base, 16)]
            s16 = src_st[pl.ds(base, 16)]
            w16 = w_st[pl.ds(base, 16)]
            dl = d16 - row0
            ok = (dl >= 0) & (dl < nrows)
            pk = jnp.where(ok, (dl << _SHIFT) | s16, 0)
            cs = plsc.cumsum(jnp.where(ok, 1, 0))
            pos = off + cs - 1
            plsc.store_scatter(pend_pk, [pos], pk, mask=ok)
            plsc.store_scatter(pend_w, [pos], w16, mask=ok)
            return off + cs[15]

        cnt = lax.fori_loop(0, _NG, scan, jnp.int32(0))

        def chunk(i, carry2):
            cb = i * _K
            for q in range(_K // 16):
                qb = cb + q * 16
                valid = (qb + iota16) < cnt
                pk = jnp.where(valid, pend_pk[pl.ds(qb, 16)], 0)
                wv = jnp.where(valid, pend_w[pl.ds(qb, 16)], 0.0)
                srcb[pl.ds(q * 16, 16)] = pk & _MASKLO
                dlb[pl.ds(q * 16, 16)] = pk >> _SHIFT
                wb[pl.ds(q * 16, 16)] = wv
            pltpu.async_copy(h_hbm.at[srcb], rows_v, sem).wait()
            for q in range(_K // 16):
                dlv = dlb[pl.ds(q * 16, 16)]
                wv = wb[pl.ds(q * 16, 16)]
                for lane in range(16):
                    r = dlv[lane]
                    we = wv[lane]
                    e = q * 16 + lane
                    for j in range(_D // 16):
                        plsc.addupdate(
                            acc.at[r, pl.ds(j * 16, 16)],
                            rows_v[e, pl.ds(j * 16, 16)] * we,
                        )
            return carry2

        nch = (cnt + (_K - 1)) // _K
        lax.fori_loop(0, nch, chunk, 0)
        return carry

    lax.fori_loop(0, _NB, block, 0)

    @pl.when(g < 2)
    def _():
        pltpu.sync_copy(acc.at[pl.ds(0, _RA)], out_hbm.at[pl.ds(row0, _RA)])

    @pl.when(g >= 2)
    def _():
        pltpu.sync_copy(acc.at[pl.ds(0, _RB)], out_hbm.at[pl.ds(row0, _RB)])


def _spmm(h, src, dst, w):
    mesh = plsc.VectorSubcoreMesh(core_axis_name="c", subcore_axis_name="s")
    fn = pl.kernel(
        _spmm_body,
        out_type=jax.ShapeDtypeStruct((_N, _D), jnp.float32),
        mesh=mesh,
        compiler_params=pltpu.CompilerParams(needs_layout_passes=False),
        scratch_types=[
            pltpu.VMEM((_SB,), jnp.int32),
            pltpu.VMEM((_SB,), jnp.int32),
            pltpu.VMEM((_SB,), jnp.float32),
            pltpu.VMEM((_PEND,), jnp.int32),
            pltpu.VMEM((_PEND,), jnp.float32),
            pltpu.VMEM((_K,), jnp.int32),
            pltpu.VMEM((_K,), jnp.int32),
            pltpu.VMEM((_K,), jnp.float32),
            pltpu.VMEM((_K, _D), jnp.float32),
            pltpu.VMEM((_RA, _D), jnp.float32),
            pltpu.SemaphoreType.DMA,
        ],
    )
    return fn(h, src, dst, w)


def kernel(X, edge_index, edge_weight, W1, b1, a, W2, b2):
    src = edge_index[0]
    dst = edge_index[1]
    h1 = _mm(X, W1, b1, a, prelu=False)
    s1 = _spmm(h1, src, dst, edge_weight)
    h2 = _mm(s1, W2, b2, a, prelu=True)
    out = _spmm(h2, src, dst, edge_weight)
    return out


# speedup vs baseline: 1.0100x; 1.0100x over previous
import functools

import jax
import jax.numpy as jnp
from jax import lax
from jax.experimental import pallas as pl
from jax.experimental.pallas import tpu as pltpu
from jax.experimental.pallas import tpu_sc as plsc

_N = 10000
_E = 160000
_D = 256
_NT = 32
_RA = 320
_RB = 312
_SB = 2000
_NB = _E // _SB
_NG = _SB // 16
_K = 80
_PEND = _SB + 16
_SHIFT = 14
_MASKLO = (1 << _SHIFT) - 1


def _mm_body(a_ref, x_ref, w_ref, b_ref, o_ref, *, prelu):
    x = x_ref[...]
    if prelu:
        al = a_ref[0]
        x = jnp.where(x >= 0, x, al * x)
    o_ref[...] = (
        jnp.dot(x, w_ref[...], preferred_element_type=jnp.float32) + b_ref[...]
    )


def _mm(x, w, b, a, prelu):
    blk = 2000
    grid = (_N // blk,)
    return pl.pallas_call(
        functools.partial(_mm_body, prelu=prelu),
        grid=grid,
        in_specs=[
            pl.BlockSpec(memory_space=pltpu.SMEM),
            pl.BlockSpec((blk, _D), lambda i: (i, 0)),
            pl.BlockSpec((_D, _D), lambda i: (0, 0)),
            pl.BlockSpec((1, _D), lambda i: (0, 0)),
        ],
        out_specs=pl.BlockSpec((blk, _D), lambda i: (i, 0)),
        out_shape=jax.ShapeDtypeStruct((_N, _D), jnp.float32),
    )(a, x, w, b.reshape(1, _D))


def _spmm_body(h_hbm, src_hbm, dst_hbm, w_hbm, out_hbm,
               src0, dst0, w0, src1, dst1, w1,
               pk0, pw0, pk1, pw1,
               srcb, dlb, wb, rows_v, acc,
               ssem0, ssem1, gsem):
    c = lax.axis_index("c")
    s = lax.axis_index("s")
    g = c * 16 + s
    row0 = g * _RB + jnp.minimum(g, 2) * (_RA - _RB)
    nrows = jnp.where(g < 2, _RA, _RB)

    stage_bufs = [(src0, dst0, w0), (src1, dst1, w1)]
    pend_bufs = [(pk0, pw0), (pk1, pw1)]
    ssems = [ssem0, ssem1]

    def _zero(r, carry):
        for j in range(_D // 16):
            acc[r, pl.ds(j * 16, 16)] = jnp.zeros((16,), jnp.float32)
        return carry

    lax.fori_loop(0, _RA, _zero, 0)

    iota16 = lax.iota(jnp.int32, 16)

    def stage_start(b, p):
        eb = b * _SB
        S, sem = stage_bufs[p], ssems[p]
        pltpu.async_copy(src_hbm.at[pl.ds(eb, _SB)], S[0], sem)
        pltpu.async_copy(dst_hbm.at[pl.ds(eb, _SB)], S[1], sem)
        pltpu.async_copy(w_hbm.at[pl.ds(eb, _SB)], S[2], sem)

    def stage_wait(p):
        S, sem = stage_bufs[p], ssems[p]
        pltpu.make_async_copy(src_hbm.at[pl.ds(0, _SB)], S[0], sem).wait()
        pltpu.make_async_copy(dst_hbm.at[pl.ds(0, _SB)], S[1], sem).wait()
        pltpu.make_async_copy(w_hbm.at[pl.ds(0, _SB)], S[2], sem).wait()

    def scan_block(p):
        (src_st, dst_st, w_st), (pend_pk, pend_w) = stage_bufs[p], pend_bufs[p]

        def scan(i, off):
            base = i * 16
            d16 = dst_st[pl.ds(base, 16)]
            s16 = src_st[pl.ds(base, 16)]
            w16 = w_st[pl.ds(base, 16)]
            dl = d16 - row0
            ok = (dl >= 0) & (dl < nrows)
            pk = jnp.where(ok, (dl << _SHIFT) | s16, 0)
            cs = plsc.cumsum(jnp.where(ok, 1, 0))
            pos = off + cs - 1
            plsc.store_scatter(pend_pk, [pos], pk, mask=ok)
            plsc.store_scatter(pend_w, [pos], w16, mask=ok)
            return off + cs[15]

        return lax.fori_loop(0, _NG, scan, jnp.int32(0))

    def prep_chunk(p, ci, cnt):
        pend_pk, pend_w = pend_bufs[p]
        cb = ci * _K
        for q in range(_K // 16):
            qb = cb + q * 16
            valid = (qb + iota16) < cnt
            pk = jnp.where(valid, pend_pk[pl.ds(qb, 16)], 0)
            wv = jnp.where(valid, pend_w[pl.ds(qb, 16)], 0.0)
            srcb[pl.ds(q * 16, 16)] = pk & _MASKLO
            dlb[pl.ds(q * 16, 16)] = pk >> _SHIFT
            wb[pl.ds(q * 16, 16)] = wv

    def gather_start():
        pltpu.async_copy(h_hbm.at[srcb], rows_v, gsem)

    def gather_wait():
        pltpu.make_async_copy(h_hbm.at[srcb], rows_v, gsem).wait()

    def accum_chunk():
        def accum_q(q, carry):
            dlv = dlb[pl.ds(q * 16, 16)]
            wv = wb[pl.ds(q * 16, 16)]
            for lane in range(16):
                r = dlv[lane]
                we = wv[lane]
                e = q * 16 + lane
                vals = [rows_v[e, pl.ds(j * 16, 16)] for j in range(_D // 16)]
                for j in range(_D // 16):
                    plsc.addupdate(acc.at[r, pl.ds(j * 16, 16)], vals[j] * we)
            return carry

        lax.fori_loop(0, _K // 16, accum_q, 0)

    def extra_chunks(p, cnt):
        nch = (cnt + (_K - 1)) // _K

        def chunk(i, carry2):
            prep_chunk(p, i, cnt)
            gather_start()
            gather_wait()
            accum_chunk()
            return carry2

        lax.fori_loop(1, nch, chunk, 0)

    def body(b, pb, cnt_b, stage_next=True):
        stage_wait(1 - pb)
        cnt_n = scan_block(1 - pb)
        if stage_next:
            stage_start(b + 2, pb)
        gather_wait()
        accum_chunk()
        extra_chunks(pb, cnt_b)
        prep_chunk(1 - pb, 0, cnt_n)
        gather_start()
        return cnt_n

    stage_start(0, 0)
    stage_wait(0)
    cnt = scan_block(0)
    stage_start(1, 1)
    prep_chunk(0, 0, cnt)
    gather_start()

    def pair(i, cnt_c):
        b = i * 2
        cnt1 = body(b, 0, cnt_c)
        cnt2 = body(b + 1, 1, cnt1)
        return cnt2

    cnt = lax.fori_loop(0, (_NB - 2) // 2, pair, cnt)
    cnt_last = body(_NB - 2, 0, cnt, stage_next=False)
    gather_wait()
    accum_chunk()
    extra_chunks(1, cnt_last)

    @pl.when(g < 2)
    def _():
        pltpu.sync_copy(acc.at[pl.ds(0, _RA)], out_hbm.at[pl.ds(row0, _RA)])

    @pl.when(g >= 2)
    def _():
        pltpu.sync_copy(acc.at[pl.ds(0, _RB)], out_hbm.at[pl.ds(row0, _RB)])


def _spmm(h, src, dst, w):
    mesh = plsc.VectorSubcoreMesh(core_axis_name="c", subcore_axis_name="s")
    fn = pl.kernel(
        _spmm_body,
        out_type=jax.ShapeDtypeStruct((_N, _D), jnp.float32),
        mesh=mesh,
        compiler_params=pltpu.CompilerParams(needs_layout_passes=False),
        scratch_types=[
            pltpu.VMEM((_SB,), jnp.int32),
            pltpu.VMEM((_SB,), jnp.int32),
            pltpu.VMEM((_SB,), jnp.float32),
            pltpu.VMEM((_SB,), jnp.int32),
            pltpu.VMEM((_SB,), jnp.int32),
            pltpu.VMEM((_SB,), jnp.float32),
            pltpu.VMEM((_PEND,), jnp.int32),
            pltpu.VMEM((_PEND,), jnp.float32),
            pltpu.VMEM((_PEND,), jnp.int32),
            pltpu.VMEM((_PEND,), jnp.float32),
            pltpu.VMEM((_K,), jnp.int32),
            pltpu.VMEM((_K,), jnp.int32),
            pltpu.VMEM((_K,), jnp.float32),
            pltpu.VMEM((_K, _D), jnp.float32),
            pltpu.VMEM((_RA, _D), jnp.float32),
            pltpu.SemaphoreType.DMA,
            pltpu.SemaphoreType.DMA,
            pltpu.SemaphoreType.DMA,
        ],
    )
    return fn(h, src, dst, w)


def kernel(X, edge_index, edge_weight, W1, b1, a, W2, b2):
    src = edge_index[0]
    dst = edge_index[1]
    h1 = _mm(X, W1, b1, a, prelu=False)
    s1 = _spmm(h1, src, dst, edge_weight)
    h2 = _mm(s1, W2, b2, a, prelu=True)
    out = _spmm(h2, src, dst, edge_weight)
    return out


# speedup vs baseline: 1.0101x; 1.0001x over previous
import functools

import jax
import jax.numpy as jnp
from jax import lax
from jax.experimental import pallas as pl
from jax.experimental.pallas import tpu as pltpu
from jax.experimental.pallas import tpu_sc as plsc

_N = 10000
_E = 160000
_D = 256
_NT = 32
_RA = 320
_RB = 312
_SB = 2000
_NB = _E // _SB
_NG = _SB // 16
_K = 80
_PEND = _SB + 16
_SHIFT = 14
_MASKLO = (1 << _SHIFT) - 1


def _mm_body(a_ref, x_ref, w_ref, b_ref, o_ref, *, prelu):
    x = x_ref[...]
    if prelu:
        al = a_ref[0]
        x = jnp.where(x >= 0, x, al * x)
    o_ref[...] = (
        jnp.dot(x, w_ref[...], preferred_element_type=jnp.float32) + b_ref[...]
    )


def _mm(x, w, b, a, prelu):
    blk = 2000
    grid = (_N // blk,)
    return pl.pallas_call(
        functools.partial(_mm_body, prelu=prelu),
        grid=grid,
        in_specs=[
            pl.BlockSpec(memory_space=pltpu.SMEM),
            pl.BlockSpec((blk, _D), lambda i: (i, 0)),
            pl.BlockSpec((_D, _D), lambda i: (0, 0)),
            pl.BlockSpec((1, _D), lambda i: (0, 0)),
        ],
        out_specs=pl.BlockSpec((blk, _D), lambda i: (i, 0)),
        out_shape=jax.ShapeDtypeStruct((_N, _D), jnp.float32),
    )(a, x, w, b.reshape(1, _D))


def _spmm_body(h_hbm, src_hbm, dst_hbm, w_hbm, out_hbm,
               src0, dst0, w0, src1, dst1, w1,
               pk0, pw0, pk1, pw1,
               srcb, dlb, wb, rows_v, acc,
               ssem0, ssem1, gsem):
    c = lax.axis_index("c")
    s = lax.axis_index("s")
    g = c * 16 + s
    row0 = g * _RB + jnp.minimum(g, 2) * (_RA - _RB)
    nrows = jnp.where(g < 2, _RA, _RB)

    stage_bufs = [(src0, dst0, w0), (src1, dst1, w1)]
    pend_bufs = [(pk0, pw0), (pk1, pw1)]
    ssems = [ssem0, ssem1]

    def _zero(r, carry):
        for j in range(_D // 16):
            acc[r, pl.ds(j * 16, 16)] = jnp.zeros((16,), jnp.float32)
        return carry

    lax.fori_loop(0, _RA, _zero, 0)

    iota16 = lax.iota(jnp.int32, 16)

    def stage_start(b, p):
        eb = b * _SB
        S, sem = stage_bufs[p], ssems[p]
        pltpu.async_copy(src_hbm.at[pl.ds(eb, _SB)], S[0], sem)
        pltpu.async_copy(dst_hbm.at[pl.ds(eb, _SB)], S[1], sem)
        pltpu.async_copy(w_hbm.at[pl.ds(eb, _SB)], S[2], sem)

    def stage_wait(p):
        S, sem = stage_bufs[p], ssems[p]
        pltpu.make_async_copy(src_hbm.at[pl.ds(0, _SB)], S[0], sem).wait()
        pltpu.make_async_copy(dst_hbm.at[pl.ds(0, _SB)], S[1], sem).wait()
        pltpu.make_async_copy(w_hbm.at[pl.ds(0, _SB)], S[2], sem).wait()

    def scan_block(p):
        (src_st, dst_st, w_st), (pend_pk, pend_w) = stage_bufs[p], pend_bufs[p]

        def scan(i, off):
            base = i * 16
            d16 = dst_st[pl.ds(base, 16)]
            s16 = src_st[pl.ds(base, 16)]
            w16 = w_st[pl.ds(base, 16)]
            dl = d16 - row0
            ok = (dl >= 0) & (dl < nrows)
            pk = jnp.where(ok, (dl << _SHIFT) | s16, 0)
            cs = plsc.cumsum(jnp.where(ok, 1, 0))
            pos = off + cs - 1
            plsc.store_scatter(pend_pk, [pos], pk, mask=ok)
            plsc.store_scatter(pend_w, [pos], w16, mask=ok)
            return off + cs[15]

        return lax.fori_loop(0, _NG, scan, jnp.int32(0))

    def prep_chunk(p, ci, cnt):
        pend_pk, pend_w = pend_bufs[p]
        cb = ci * _K
        for q in range(_K // 16):
            qb = cb + q * 16
            valid = (qb + iota16) < cnt
            pk = jnp.where(valid, pend_pk[pl.ds(qb, 16)], 0)
            wv = jnp.where(valid, pend_w[pl.ds(qb, 16)], 0.0)
            srcb[pl.ds(q * 16, 16)] = pk & _MASKLO
            dlb[pl.ds(q * 16, 16)] = pk >> _SHIFT
            wb[pl.ds(q * 16, 16)] = wv

    def gather_start():
        pltpu.async_copy(h_hbm.at[srcb], rows_v, gsem)

    def gather_wait():
        pltpu.make_async_copy(h_hbm.at[srcb], rows_v, gsem).wait()

    def accum_chunk():
        def accum_q(q, carry):
            dlv = dlb[pl.ds(q * 16, 16)]
            wv = wb[pl.ds(q * 16, 16)]
            for lane in range(16):
                r = dlv[lane]
                we = wv[lane]
                e = q * 16 + lane
                vals = [rows_v[e, pl.ds(j * 16, 16)] for j in range(_D // 16)]
                for j in range(_D // 16):
                    plsc.addupdate(acc.at[r, pl.ds(j * 16, 16)], vals[j] * we)
            return carry

        lax.fori_loop(0, _K // 16, accum_q, 0)

    def extra_chunks(p, cnt):
        nch = (cnt + (_K - 1)) // _K

        def chunk(i, carry2):
            prep_chunk(p, i, cnt)
            gather_start()
            gather_wait()
            accum_chunk()
            return carry2

        lax.fori_loop(1, nch, chunk, 0)

    def body(b, pb, cnt_b, stage_next=True):
        with jax.named_scope("stwait"):
            stage_wait(1 - pb)
        with jax.named_scope("scan"):
            cnt_n = scan_block(1 - pb)
        if stage_next:
            stage_start(b + 2, pb)
        with jax.named_scope("gwait"):
            gather_wait()
        with jax.named_scope("accum"):
            accum_chunk()
        with jax.named_scope("extra"):
            extra_chunks(pb, cnt_b)
        with jax.named_scope("prep"):
            prep_chunk(1 - pb, 0, cnt_n)
        gather_start()
        return cnt_n

    stage_start(0, 0)
    stage_wait(0)
    cnt = scan_block(0)
    stage_start(1, 1)
    prep_chunk(0, 0, cnt)
    gather_start()

    def pair(i, cnt_c):
        b = i * 2
        cnt1 = body(b, 0, cnt_c)
        cnt2 = body(b + 1, 1, cnt1)
        return cnt2

    cnt = lax.fori_loop(0, (_NB - 2) // 2, pair, cnt)
    cnt_last = body(_NB - 2, 0, cnt, stage_next=False)
    gather_wait()
    accum_chunk()
    extra_chunks(1, cnt_last)

    @pl.when(g < 2)
    def _():
        pltpu.sync_copy(acc.at[pl.ds(0, _RA)], out_hbm.at[pl.ds(row0, _RA)])

    @pl.when(g >= 2)
    def _():
        pltpu.sync_copy(acc.at[pl.ds(0, _RB)], out_hbm.at[pl.ds(row0, _RB)])


def _spmm(h, src, dst, w):
    mesh = plsc.VectorSubcoreMesh(core_axis_name="c", subcore_axis_name="s")
    fn = pl.kernel(
        _spmm_body,
        out_type=jax.ShapeDtypeStruct((_N, _D), jnp.float32),
        mesh=mesh,
        compiler_params=pltpu.CompilerParams(needs_layout_passes=False),
        scratch_types=[
            pltpu.VMEM((_SB,), jnp.int32),
            pltpu.VMEM((_SB,), jnp.int32),
            pltpu.VMEM((_SB,), jnp.float32),
            pltpu.VMEM((_SB,), jnp.int32),
            pltpu.VMEM((_SB,), jnp.int32),
            pltpu.VMEM((_SB,), jnp.float32),
            pltpu.VMEM((_PEND,), jnp.int32),
            pltpu.VMEM((_PEND,), jnp.float32),
            pltpu.VMEM((_PEND,), jnp.int32),
            pltpu.VMEM((_PEND,), jnp.float32),
            pltpu.VMEM((_K,), jnp.int32),
            pltpu.VMEM((_K,), jnp.int32),
            pltpu.VMEM((_K,), jnp.float32),
            pltpu.VMEM((_K, _D), jnp.float32),
            pltpu.VMEM((_RA, _D), jnp.float32),
            pltpu.SemaphoreType.DMA,
            pltpu.SemaphoreType.DMA,
            pltpu.SemaphoreType.DMA,
        ],
    )
    return fn(h, src, dst, w)


def kernel(X, edge_index, edge_weight, W1, b1, a, W2, b2):
    src = edge_index[0]
    dst = edge_index[1]
    h1 = _mm(X, W1, b1, a, prelu=False)
    s1 = _spmm(h1, src, dst, edge_weight)
    h2 = _mm(s1, W2, b2, a, prelu=True)
    out = _spmm(h2, src, dst, edge_weight)
    return out


# speedup vs baseline: 5.1582x; 5.1065x over previous
import functools

import jax
import jax.numpy as jnp
from jax import lax
from jax.experimental import pallas as pl
from jax.experimental.pallas import tpu as pltpu
from jax.experimental.pallas import tpu_sc as plsc

_N = 10000
_E = 160000
_D = 256
_NT = 32
_RA = 320
_RB = 312
_SB = 2000
_NB = _E // _SB
_NG = _SB // 16
_K = 80
_PEND = _SB + 16
_SHIFT = 14
_MASKLO = (1 << _SHIFT) - 1


def _mm_body(a_ref, x_ref, w_ref, b_ref, o_ref, *, prelu):
    x = x_ref[...]
    if prelu:
        al = a_ref[0]
        x = jnp.where(x >= 0, x, al * x)
    o_ref[...] = (
        jnp.dot(x, w_ref[...], preferred_element_type=jnp.float32) + b_ref[...]
    )


def _mm(x, w, b, a, prelu):
    blk = 2000
    grid = (_N // blk,)
    return pl.pallas_call(
        functools.partial(_mm_body, prelu=prelu),
        grid=grid,
        in_specs=[
            pl.BlockSpec(memory_space=pltpu.SMEM),
            pl.BlockSpec((blk, _D), lambda i: (i, 0)),
            pl.BlockSpec((_D, _D), lambda i: (0, 0)),
            pl.BlockSpec((1, _D), lambda i: (0, 0)),
        ],
        out_specs=pl.BlockSpec((blk, _D), lambda i: (i, 0)),
        out_shape=jax.ShapeDtypeStruct((_N, _D), jnp.float32),
    )(a, x, w, b.reshape(1, _D))


def _spmm_body(h_hbm, src_hbm, dst_hbm, w_hbm, out_hbm,
               src0, dst0, w0, src1, dst1, w1,
               pk0, pw0, pk1, pw1,
               srcb, dlb, wb, rows_v, acc,
               ssem0, ssem1, gsem):
    c = lax.axis_index("c")
    s = lax.axis_index("s")
    g = c * 16 + s
    row0 = g * _RB + jnp.minimum(g, 2) * (_RA - _RB)
    nrows = jnp.where(g < 2, _RA, _RB)

    stage_bufs = [(src0, dst0, w0), (src1, dst1, w1)]
    pend_bufs = [(pk0, pw0), (pk1, pw1)]
    ssems = [ssem0, ssem1]

    def _zero(r, carry):
        for j in range(_D // 16):
            acc[r, pl.ds(j * 16, 16)] = jnp.zeros((16,), jnp.float32)
        return carry

    lax.fori_loop(0, _RA, _zero, 0)

    iota16 = lax.iota(jnp.int32, 16)

    def stage_start(b, p):
        eb = b * _SB
        S, sem = stage_bufs[p], ssems[p]
        pltpu.async_copy(src_hbm.at[pl.ds(eb, _SB)], S[0], sem)
        pltpu.async_copy(dst_hbm.at[pl.ds(eb, _SB)], S[1], sem)
        pltpu.async_copy(w_hbm.at[pl.ds(eb, _SB)], S[2], sem)

    def stage_wait(p):
        S, sem = stage_bufs[p], ssems[p]
        pltpu.make_async_copy(src_hbm.at[pl.ds(0, _SB)], S[0], sem).wait()
        pltpu.make_async_copy(dst_hbm.at[pl.ds(0, _SB)], S[1], sem).wait()
        pltpu.make_async_copy(w_hbm.at[pl.ds(0, _SB)], S[2], sem).wait()

    def scan_block(p):
        (src_st, dst_st, w_st), (pend_pk, pend_w) = stage_bufs[p], pend_bufs[p]

        def one_group(base, off):
            d16 = dst_st[pl.ds(base, 16)]
            s16 = src_st[pl.ds(base, 16)]
            w16 = w_st[pl.ds(base, 16)]
            dl = d16 - row0
            ok = (dl >= 0) & (dl < nrows)
            pk = jnp.where(ok, (dl << _SHIFT) | s16, 0)
            cs = plsc.cumsum(jnp.where(ok, 1, 0))
            pos = off + cs - 1
            plsc.store_scatter(pend_pk, [pos], pk, mask=ok)
            plsc.store_scatter(pend_w, [pos], w16, mask=ok)
            return off + cs[15]

        def scan2(i, off):
            base = i * 32
            off = one_group(base, off)
            return one_group(base + 16, off)

        off = lax.fori_loop(0, _NG // 2, scan2, jnp.int32(0))
        return one_group((_NG - 1) * 16, off)

    def prep_chunk(p, ci, cnt):
        pend_pk, pend_w = pend_bufs[p]
        cb = ci * _K
        for q in range(_K // 16):
            qb = cb + q * 16
            valid = (qb + iota16) < cnt
            pk = pend_pk[pl.ds(qb, 16)]
            wv = jnp.where(valid, pend_w[pl.ds(qb, 16)], 0.0)
            pad_src = g * _K + q * 16 + iota16
            srcb[pl.ds(q * 16, 16)] = jnp.where(valid, pk & _MASKLO, pad_src)
            dlb[pl.ds(q * 16, 16)] = jnp.where(valid, pk >> _SHIFT, 0)
            wb[pl.ds(q * 16, 16)] = wv

    def gather_start():
        for q in range(_K // 16):
            sv = srcb[pl.ds(q * 16, 16)]
            for lane in range(16):
                si = sv[lane]
                e = q * 16 + lane
                pltpu.async_copy(h_hbm.at[pl.ds(si * _D, _D)],
                                 rows_v.at[pl.ds(e * _D, _D)], gsem)

    def gather_wait():
        pltpu.make_async_copy(h_hbm.at[pl.ds(0, _K * _D)], rows_v, gsem).wait()

    def accum_chunk():
        def accum_q(q, carry):
            dlv = dlb[pl.ds(q * 16, 16)]
            wv = wb[pl.ds(q * 16, 16)]
            prev = None
            for lane in range(16):
                we = wv[lane]
                e = q * 16 + lane
                vals = [rows_v[pl.ds(e * _D + j * 16, 16)] * we
                        for j in range(_D // 16)]
                if prev is not None:
                    pr, pvals = prev
                    for j in range(_D // 16):
                        plsc.addupdate(acc.at[pr, pl.ds(j * 16, 16)],
                                       pvals[j])
                prev = (dlv[lane], vals)
            pr, pvals = prev
            for j in range(_D // 16):
                plsc.addupdate(acc.at[pr, pl.ds(j * 16, 16)], pvals[j])
            return carry

        lax.fori_loop(0, _K // 16, accum_q, 0)

    def extra_chunks(p, cnt):
        nch = (cnt + (_K - 1)) // _K

        def chunk(i, carry2):
            prep_chunk(p, i, cnt)
            gather_start()
            gather_wait()
            accum_chunk()
            return carry2

        lax.fori_loop(1, nch, chunk, 0)

    def body(b, pb, cnt_b, stage_next=True):
        stage_wait(1 - pb)
        cnt_n = scan_block(1 - pb)
        if stage_next:
            stage_start(b + 2, pb)
        gather_wait()
        accum_chunk()
        extra_chunks(pb, cnt_b)
        prep_chunk(1 - pb, 0, cnt_n)
        gather_start()
        return cnt_n

    stage_start(0, 0)
    stage_wait(0)
    cnt = scan_block(0)
    stage_start(1, 1)
    prep_chunk(0, 0, cnt)
    gather_start()

    def pair(i, cnt_c):
        b = i * 2
        cnt1 = body(b, 0, cnt_c)
        cnt2 = body(b + 1, 1, cnt1)
        return cnt2

    cnt = lax.fori_loop(0, (_NB - 2) // 2, pair, cnt)
    cnt_last = body(_NB - 2, 0, cnt, stage_next=False)
    gather_wait()
    accum_chunk()
    extra_chunks(1, cnt_last)

    @pl.when(g < 2)
    def _():
        pltpu.sync_copy(acc.at[pl.ds(0, _RA)], out_hbm.at[pl.ds(row0, _RA)])

    @pl.when(g >= 2)
    def _():
        pltpu.sync_copy(acc.at[pl.ds(0, _RB)], out_hbm.at[pl.ds(row0, _RB)])


def _spmm(h, src, dst, w):
    h = h.reshape(_N * _D)
    mesh = plsc.VectorSubcoreMesh(core_axis_name="c", subcore_axis_name="s")
    fn = pl.kernel(
        _spmm_body,
        out_type=jax.ShapeDtypeStruct((_N, _D), jnp.float32),
        mesh=mesh,
        compiler_params=pltpu.CompilerParams(needs_layout_passes=False),
        scratch_types=[
            pltpu.VMEM((_SB,), jnp.int32),
            pltpu.VMEM((_SB,), jnp.int32),
            pltpu.VMEM((_SB,), jnp.float32),
            pltpu.VMEM((_SB,), jnp.int32),
            pltpu.VMEM((_SB,), jnp.int32),
            pltpu.VMEM((_SB,), jnp.float32),
            pltpu.VMEM((_PEND,), jnp.int32),
            pltpu.VMEM((_PEND,), jnp.float32),
            pltpu.VMEM((_PEND,), jnp.int32),
            pltpu.VMEM((_PEND,), jnp.float32),
            pltpu.VMEM((_K,), jnp.int32),
            pltpu.VMEM((_K,), jnp.int32),
            pltpu.VMEM((_K,), jnp.float32),
            pltpu.VMEM((_K * _D,), jnp.float32),
            pltpu.VMEM((_RA, _D), jnp.float32),
            pltpu.SemaphoreType.DMA,
            pltpu.SemaphoreType.DMA,
            pltpu.SemaphoreType.DMA,
        ],
    )
    return fn(h, src, dst, w)


def kernel(X, edge_index, edge_weight, W1, b1, a, W2, b2):
    src = edge_index[0]
    dst = edge_index[1]
    h1 = _mm(X, W1, b1, a, prelu=False)
    s1 = _spmm(h1, src, dst, edge_weight)
    h2 = _mm(s1, W2, b2, a, prelu=True)
    out = _spmm(h2, src, dst, edge_weight)
    return out
